# Initial kernel scaffold; baseline (speedup 1.0000x reference)
import jax
import jax.numpy as jnp
from jax.experimental import pallas as pl


def kernel(x, edge_index, W_in, b_in, bn1_g, bn1_b, Wl1, bl1, Wr1, Wl2, bl2, Wr2, Ws, bs, bn2_g, bn2_b):
    raise NotImplementedError("write your pallas kernel here")


# speedup vs baseline: 4.9332x; 4.9332x over previous
import functools

import jax
import jax.numpy as jnp
from jax import lax
from jax.experimental import pallas as pl
from jax.experimental.pallas import tpu as pltpu
from jax.experimental.pallas import tpu_sc as plsc

D = 128
NC = 2
NS = 16
NW = NC * NS
L = 16


def _make_sc_agg(n, e, with_cnt):
    epw = e // NW
    C = 80
    n_chunks = epw // C
    assert n_chunks * C == epw
    rpt = (n // NS) // 8 * 8
    nrem = n - rpt * NS
    ZR = 24
    assert rpt % ZR == 0 and nrem % 8 == 0 and nrem <= ZR

    mesh = plsc.VectorSubcoreMesh(core_axis_name="c", subcore_axis_name="s",
                                  num_cores=NC, num_subcores=NS)

    out_type = [jax.ShapeDtypeStruct((NC, n, D), jnp.float32)]
    scratch = [
        pltpu.VMEM_SHARED((n, D), jnp.float32),
        pltpu.VMEM((C,), jnp.int32),
        pltpu.VMEM((C,), jnp.int32),
        pltpu.VMEM((C, D), jnp.float32),
        pltpu.VMEM((ZR, D), jnp.float32),
        pltpu.SemaphoreType.DMA,
    ]
    if with_cnt:
        out_type.append(jax.ShapeDtypeStruct((NW, n), jnp.float32))
        scratch.append(pltpu.VMEM((n,), jnp.float32))

    @functools.partial(pl.kernel, mesh=mesh,
                       out_type=tuple(out_type) if with_cnt else out_type[0],
                       scratch_types=scratch,
                       compiler_params=pltpu.CompilerParams(
                           needs_layout_passes=False))
    def sc_agg(h_hbm, src_hbm, dst_hbm, out_hbm, *rest):
        if with_cnt:
            cnt_hbm, acc, srcv, dstv, rows, ztile, sem, cntv = rest
        else:
            acc, srcv, dstv, rows, ztile, sem = rest
        cid = lax.axis_index("c")
        sid = lax.axis_index("s")
        wid = sid * NC + cid

        def zero_row(i, carry):
            for j in range(D // L):
                ztile[i, pl.ds(j * L, L)] = jnp.zeros((L,), jnp.float32)
            return carry
        lax.fori_loop(0, ZR, zero_row, 0)

        def zero_fill(k, carry):
            pltpu.sync_copy(ztile, acc.at[pl.ds(sid * rpt + k * ZR, ZR)])
            return carry
        lax.fori_loop(0, rpt // ZR, zero_fill, 0)
        if nrem:
            @pl.when(sid == 0)
            def _():
                pltpu.sync_copy(ztile.at[pl.ds(0, nrem)],
                                acc.at[pl.ds(NS * rpt, nrem)])

        if with_cnt:
            def zero_cnt(i, carry):
                cntv[pl.ds(i * L, L)] = jnp.zeros((L,), jnp.float32)
                return carry
            lax.fori_loop(0, n // L, zero_cnt, 0)

        plsc.subcore_barrier()

        base = wid * epw
        ones = jnp.ones((L,), jnp.float32)

        def edge_chunk(g, carry):
            off = base + g * C
            pltpu.sync_copy(src_hbm.at[pl.ds(off, C)], srcv)
            pltpu.sync_copy(dst_hbm.at[pl.ds(off, C)], dstv)
            pltpu.async_copy(h_hbm.at[srcv], rows, sem).wait()
            pltpu.sync_copy(rows, acc.at[dstv], add=True)
            if with_cnt:
                for k in range(C // L):
                    idx = dstv[pl.ds(k * L, L)]
                    plsc.addupdate_scatter(cntv, [idx], ones)
            return carry
        lax.fori_loop(0, n_chunks, edge_chunk, 0)
        plsc.subcore_barrier()

        pltpu.sync_copy(acc.at[pl.ds(sid * rpt, rpt)],
                        out_hbm.at[cid, pl.ds(sid * rpt, rpt)])
        if nrem:
            @pl.when(sid == 0)
            def _():
                pltpu.sync_copy(acc.at[pl.ds(NS * rpt, nrem)],
                                out_hbm.at[cid, pl.ds(NS * rpt, nrem)])
        if with_cnt:
            pltpu.sync_copy(cntv, cnt_hbm.at[wid])

    return sc_agg


def _mm_t(a, w):
    return lax.dot_general(a, w, (((1,), (1,)), ((), ())),
                           precision=lax.Precision.HIGHEST,
                           preferred_element_type=jnp.float32)


def _leaky(h):
    return jnp.where(h > 0, h, 0.2 * h)


def _bn(h, g, b):
    mu = jnp.mean(h, axis=0, keepdims=True)
    var = jnp.mean((h - mu) ** 2, axis=0, keepdims=True)
    return g * (h - mu) * lax.rsqrt(var + 1e-5) + b


def _tc_input_body(x_ref, w_ref, b_ref, out_ref):
    out_ref[...] = _leaky(_mm_t(x_ref[...], w_ref[...]) + b_ref[...])


def _tc_bn1_body(h_ref, g_ref, bb_ref, out_ref):
    out_ref[...] = _bn(h_ref[...], g_ref[...], bb_ref[...])


def _tc_cnt_body(cntp_ref, cnt_ref):
    cnt_ref[...] = jnp.clip(jnp.sum(cntp_ref[...], axis=0), 1.0, None)[:, None]


def _tc_combine1_body(part_ref, cnt_ref, h_ref, wl_ref, bl_ref, wr_ref,
                      h1_ref):
    s = part_ref[0] + part_ref[1]
    agg = s / cnt_ref[...]
    h1_ref[...] = _leaky(_mm_t(agg, wl_ref[...]) + bl_ref[...]
                         + _mm_t(h_ref[...], wr_ref[...]))


def _tc_combine2_body(part_ref, cnt_ref, h1_ref, h_ref, wl_ref, bl_ref,
                      wr_ref, ws_ref, bs_ref, out_ref):
    s = part_ref[0] + part_ref[1]
    agg = s / cnt_ref[...]
    h2 = _mm_t(agg, wl_ref[...]) + bl_ref[...] + _mm_t(h1_ref[...], wr_ref[...])
    out_ref[...] = h2 + _mm_t(h_ref[...], ws_ref[...]) + bs_ref[...]


def _tc_bn2_body(pre_ref, g_ref, bb_ref, out_ref):
    out = _bn(pre_ref[...], g_ref[...], bb_ref[...])
    nrm = jnp.sqrt(jnp.sum(out * out, axis=1, keepdims=True))
    out_ref[...] = out / jnp.maximum(nrm, 1e-12)


def kernel(x, edge_index, W_in, b_in, bn1_g, bn1_b, Wl1, bl1, Wr1, Wl2, bl2,
           Wr2, Ws, bs, bn2_g, bn2_b):
    n, _ = x.shape
    e = edge_index.shape[1]

    src = edge_index[0].astype(jnp.int32)
    dst = edge_index[1].astype(jnp.int32)
    row = lambda v: v.reshape(1, -1)
    tc_params = pltpu.CompilerParams(vmem_limit_bytes=100 * 1024 * 1024)
    BN = 2000
    grid = (n // BN,)
    assert grid[0] * BN == n

    mat = lambda: pl.BlockSpec((BN, D), lambda i: (i, 0))
    full = lambda *s: pl.BlockSpec(s, lambda i: (0,) * len(s))
    col = lambda: pl.BlockSpec((BN, 1), lambda i: (i, 0))

    h_pre = pl.pallas_call(
        _tc_input_body,
        grid=grid,
        in_specs=[mat(), full(D, D), full(1, D)],
        out_specs=mat(),
        out_shape=jax.ShapeDtypeStruct((n, D), jnp.float32),
    )(x, W_in, row(b_in))

    h = pl.pallas_call(
        _tc_bn1_body,
        out_shape=jax.ShapeDtypeStruct((n, D), jnp.float32),
        compiler_params=tc_params,
    )(h_pre, row(bn1_g), row(bn1_b))

    part1, cntp = _make_sc_agg(n, e, True)(h, src, dst)

    cnt = pl.pallas_call(
        _tc_cnt_body,
        out_shape=jax.ShapeDtypeStruct((n, 1), jnp.float32),
    )(cntp)

    h1 = pl.pallas_call(
        _tc_combine1_body,
        grid=grid,
        in_specs=[pl.BlockSpec((NC, BN, D), lambda i: (0, i, 0)),
                  col(), mat(), full(D, D), full(1, D), full(D, D)],
        out_specs=mat(),
        out_shape=jax.ShapeDtypeStruct((n, D), jnp.float32),
    )(part1, cnt, h, Wl1, row(bl1), Wr1)

    part2 = _make_sc_agg(n, e, False)(h1, src, dst)

    pre2 = pl.pallas_call(
        _tc_combine2_body,
        grid=grid,
        in_specs=[pl.BlockSpec((NC, BN, D), lambda i: (0, i, 0)),
                  col(), mat(), mat(), full(D, D), full(1, D), full(D, D),
                  full(D, D), full(1, D)],
        out_specs=mat(),
        out_shape=jax.ShapeDtypeStruct((n, D), jnp.float32),
    )(part2, cnt, h1, h, Wl2, row(bl2), Wr2, Ws, row(bs))

    out = pl.pallas_call(
        _tc_bn2_body,
        out_shape=jax.ShapeDtypeStruct((n, D), jnp.float32),
        compiler_params=tc_params,
    )(pre2, row(bn2_g), row(bn2_b))
    return out


# speedup vs baseline: 8.7098x; 1.7655x over previous
import functools

import jax
import jax.numpy as jnp
from jax import lax
from jax.experimental import pallas as pl
from jax.experimental.pallas import tpu as pltpu
from jax.experimental.pallas import tpu_sc as plsc

D = 128
NC = 2
NS = 16
NW = NC * NS
L = 16


def _make_sc_agg(n, e, with_cnt):
    epw = e // NW
    C = 80
    n_chunks = epw // C
    assert n_chunks * C == epw
    rpt = (n // NS) // 8 * 8
    nrem = n - rpt * NS
    ZR = 24
    assert rpt % ZR == 0 and nrem % 8 == 0 and nrem <= ZR

    mesh = plsc.VectorSubcoreMesh(core_axis_name="c", subcore_axis_name="s",
                                  num_cores=NC, num_subcores=NS)

    out_type = [jax.ShapeDtypeStruct((NC, n, D), jnp.float32)]
    scratch = [
        pltpu.VMEM_SHARED((n, D), jnp.float32),
        pltpu.VMEM((C,), jnp.int32),
        pltpu.VMEM((C,), jnp.int32),
        pltpu.VMEM((C,), jnp.int32),
        pltpu.VMEM((C,), jnp.int32),
        pltpu.VMEM((C, D), jnp.float32),
        pltpu.VMEM((C, D), jnp.float32),
        pltpu.VMEM((ZR, D), jnp.float32),
        pltpu.SemaphoreType.DMA,
        pltpu.SemaphoreType.DMA,
        pltpu.SemaphoreType.DMA,
        pltpu.SemaphoreType.DMA,
        pltpu.SemaphoreType.DMA,
        pltpu.SemaphoreType.DMA,
    ]
    if with_cnt:
        out_type.append(jax.ShapeDtypeStruct((NW, n), jnp.float32))
        scratch.append(pltpu.VMEM((n,), jnp.float32))

    @functools.partial(pl.kernel, mesh=mesh,
                       out_type=tuple(out_type) if with_cnt else out_type[0],
                       scratch_types=scratch,
                       compiler_params=pltpu.CompilerParams(
                           needs_layout_passes=False))
    def sc_agg(h_hbm, src_hbm, dst_hbm, out_hbm, *rest):
        if with_cnt:
            (cnt_hbm, acc, srcv0, srcv1, dstv0, dstv1, rows0, rows1, ztile,
             ss0, ss1, ds0, ds1, gs0, gs1, cntv) = rest
        else:
            (acc, srcv0, srcv1, dstv0, dstv1, rows0, rows1, ztile,
             ss0, ss1, ds0, ds1, gs0, gs1) = rest
        srcv, dstv, rows = [srcv0, srcv1], [dstv0, dstv1], [rows0, rows1]
        ssem, dsem, gsem = [ss0, ss1], [ds0, ds1], [gs0, gs1]
        cid = lax.axis_index("c")
        sid = lax.axis_index("s")
        wid = sid * NC + cid

        def zero_row(i, carry):
            for j in range(D // L):
                ztile[i, pl.ds(j * L, L)] = jnp.zeros((L,), jnp.float32)
            return carry
        lax.fori_loop(0, ZR, zero_row, 0)

        def zero_fill(k, carry):
            pltpu.sync_copy(ztile, acc.at[pl.ds(sid * rpt + k * ZR, ZR)])
            return carry
        lax.fori_loop(0, rpt // ZR, zero_fill, 0)
        if nrem:
            @pl.when(sid == 0)
            def _():
                pltpu.sync_copy(ztile.at[pl.ds(0, nrem)],
                                acc.at[pl.ds(NS * rpt, nrem)])

        if with_cnt:
            def zero_cnt(i, carry):
                cntv[pl.ds(i * L, L)] = jnp.zeros((L,), jnp.float32)
                return carry
            lax.fori_loop(0, n // L, zero_cnt, 0)

        plsc.subcore_barrier()

        base = wid * epw
        ones = jnp.ones((L,), jnp.float32)

        def idx_start(c, b):
            off = base + c * C
            pltpu.async_copy(src_hbm.at[pl.ds(off, C)], srcv[b], ssem[b])
            pltpu.async_copy(dst_hbm.at[pl.ds(off, C)], dstv[b], dsem[b])

        def idx_wait(b):
            pltpu.make_async_copy(src_hbm.at[pl.ds(0, C)], srcv[b],
                                  ssem[b]).wait()
            pltpu.make_async_copy(dst_hbm.at[pl.ds(0, C)], dstv[b],
                                  dsem[b]).wait()

        def gat_start(b):
            pltpu.async_copy(h_hbm.at[srcv[b]], rows[b], gsem[b])

        def gat_wait(b):
            pltpu.make_async_copy(h_hbm.at[srcv[b]], rows[b], gsem[b]).wait()

        def consume(g, b):
            pltpu.sync_copy(rows[b], acc.at[dstv[b]], add=True)
            if with_cnt:
                for k in range(C // L):
                    idx = dstv[b][pl.ds(k * L, L)]
                    plsc.addupdate_scatter(cntv, [idx], ones)

        idx_start(0, 0)
        idx_wait(0)
        gat_start(0)
        idx_start(1, 1)

        def pipelined(i, carry):
            for b in range(2):
                g = 2 * i + b
                gat_wait(b)
                idx_wait(1 - b)
                gat_start(1 - b)
                consume(g, b)
                idx_start(lax.rem(g + 2, n_chunks), b)
            return carry
        lax.fori_loop(0, (n_chunks - 1) // 2, pipelined, 0)
        gat_wait(0)
        consume(n_chunks - 1, 0)
        idx_wait(1)
        plsc.subcore_barrier()

        pltpu.sync_copy(acc.at[pl.ds(sid * rpt, rpt)],
                        out_hbm.at[cid, pl.ds(sid * rpt, rpt)])
        if nrem:
            @pl.when(sid == 0)
            def _():
                pltpu.sync_copy(acc.at[pl.ds(NS * rpt, nrem)],
                                out_hbm.at[cid, pl.ds(NS * rpt, nrem)])
        if with_cnt:
            pltpu.sync_copy(cntv, cnt_hbm.at[wid])

    return sc_agg


def _mm_t(a, w):
    return lax.dot_general(a, w, (((1,), (1,)), ((), ())),
                           precision=lax.Precision.HIGHEST,
                           preferred_element_type=jnp.float32)


def _leaky(h):
    return jnp.where(h > 0, h, 0.2 * h)


def _bn(h, g, b):
    mu = jnp.mean(h, axis=0, keepdims=True)
    var = jnp.mean((h - mu) ** 2, axis=0, keepdims=True)
    return g * (h - mu) * lax.rsqrt(var + 1e-5) + b


def _tc_input_body(x_ref, w_ref, b_ref, out_ref):
    out_ref[...] = _leaky(_mm_t(x_ref[...], w_ref[...]) + b_ref[...])


def _tc_bn1_body(h_ref, g_ref, bb_ref, out_ref):
    out_ref[...] = _bn(h_ref[...], g_ref[...], bb_ref[...])


def _tc_cnt_body(cntp_ref, cnt_ref):
    cnt_ref[...] = jnp.clip(jnp.sum(cntp_ref[...], axis=0), 1.0, None)[:, None]


def _tc_combine1_body(part_ref, cnt_ref, h_ref, wl_ref, bl_ref, wr_ref,
                      h1_ref):
    s = part_ref[0] + part_ref[1]
    agg = s / cnt_ref[...]
    h1_ref[...] = _leaky(_mm_t(agg, wl_ref[...]) + bl_ref[...]
                         + _mm_t(h_ref[...], wr_ref[...]))


def _tc_combine2_body(part_ref, cnt_ref, h1_ref, h_ref, wl_ref, bl_ref,
                      wr_ref, ws_ref, bs_ref, out_ref):
    s = part_ref[0] + part_ref[1]
    agg = s / cnt_ref[...]
    h2 = _mm_t(agg, wl_ref[...]) + bl_ref[...] + _mm_t(h1_ref[...], wr_ref[...])
    out_ref[...] = h2 + _mm_t(h_ref[...], ws_ref[...]) + bs_ref[...]


def _tc_bn2_body(pre_ref, g_ref, bb_ref, out_ref):
    out = _bn(pre_ref[...], g_ref[...], bb_ref[...])
    nrm = jnp.sqrt(jnp.sum(out * out, axis=1, keepdims=True))
    out_ref[...] = out / jnp.maximum(nrm, 1e-12)


def kernel(x, edge_index, W_in, b_in, bn1_g, bn1_b, Wl1, bl1, Wr1, Wl2, bl2,
           Wr2, Ws, bs, bn2_g, bn2_b):
    n, _ = x.shape
    e = edge_index.shape[1]

    src = edge_index[0].astype(jnp.int32)
    dst = edge_index[1].astype(jnp.int32)
    row = lambda v: v.reshape(1, -1)
    tc_params = pltpu.CompilerParams(vmem_limit_bytes=100 * 1024 * 1024)
    BN = 2000
    grid = (n // BN,)
    assert grid[0] * BN == n

    mat = lambda: pl.BlockSpec((BN, D), lambda i: (i, 0))
    full = lambda *s: pl.BlockSpec(s, lambda i: (0,) * len(s))
    col = lambda: pl.BlockSpec((BN, 1), lambda i: (i, 0))

    h_pre = pl.pallas_call(
        _tc_input_body,
        grid=grid,
        in_specs=[mat(), full(D, D), full(1, D)],
        out_specs=mat(),
        out_shape=jax.ShapeDtypeStruct((n, D), jnp.float32),
    )(x, W_in, row(b_in))

    h = pl.pallas_call(
        _tc_bn1_body,
        out_shape=jax.ShapeDtypeStruct((n, D), jnp.float32),
        compiler_params=tc_params,
    )(h_pre, row(bn1_g), row(bn1_b))

    part1, cntp = _make_sc_agg(n, e, True)(h, src, dst)

    cnt = pl.pallas_call(
        _tc_cnt_body,
        out_shape=jax.ShapeDtypeStruct((n, 1), jnp.float32),
    )(cntp)

    h1 = pl.pallas_call(
        _tc_combine1_body,
        grid=grid,
        in_specs=[pl.BlockSpec((NC, BN, D), lambda i: (0, i, 0)),
                  col(), mat(), full(D, D), full(1, D), full(D, D)],
        out_specs=mat(),
        out_shape=jax.ShapeDtypeStruct((n, D), jnp.float32),
    )(part1, cnt, h, Wl1, row(bl1), Wr1)

    part2 = _make_sc_agg(n, e, False)(h1, src, dst)

    pre2 = pl.pallas_call(
        _tc_combine2_body,
        grid=grid,
        in_specs=[pl.BlockSpec((NC, BN, D), lambda i: (0, i, 0)),
                  col(), mat(), mat(), full(D, D), full(1, D), full(D, D),
                  full(D, D), full(1, D)],
        out_specs=mat(),
        out_shape=jax.ShapeDtypeStruct((n, D), jnp.float32),
    )(part2, cnt, h1, h, Wl2, row(bl2), Wr2, Ws, row(bs))

    out = pl.pallas_call(
        _tc_bn2_body,
        out_shape=jax.ShapeDtypeStruct((n, D), jnp.float32),
        compiler_params=tc_params,
    )(pre2, row(bn2_g), row(bn2_b))
    return out
